# baseline (speedup 1.0000x reference)
import jax
import jax.numpy as jnp
from jax import lax
from jax.experimental import pallas as pl
from jax.experimental.pallas import tpu as pltpu
from jax.experimental.pallas import tpu_sc as plsc

NC = 2
NS = 16
NW = NC * NS

CHUNK = 128
NB = 8


def _make_kernel(n_rows, dim, n_chunks_w):
    n_w = n_chunks_w * CHUNK
    n_groups = n_chunks_w // NB
    mesh = plsc.VectorSubcoreMesh(
        core_axis_name="c", subcore_axis_name="s",
        num_cores=NC, num_subcores=NS)

    @pl.kernel(
        out_type=jax.ShapeDtypeStruct((n_rows, dim), jnp.float32),
        mesh=mesh,
        scratch_types=[
            pltpu.VMEM((n_chunks_w, CHUNK), jnp.int32),
            [pltpu.VMEM((CHUNK, dim), jnp.float32) for _ in range(NB)],
            [pltpu.SemaphoreType.DMA for _ in range(NB)],
            [pltpu.SemaphoreType.DMA for _ in range(NB)],
        ],
        compiler_params=pltpu.CompilerParams(use_tc_tiling_on_sc=False),
    )
    def k(idx_hbm, table_hbm, out_hbm, idx_v, rows, gsems, ssems):
        wid = lax.axis_index("s") * NC + lax.axis_index("c")
        base = wid * n_w
        pltpu.sync_copy(idx_hbm.at[wid], idx_v)

        def group(g, carry):
            for b in range(NB):
                j = g * NB + b
                pltpu.make_async_copy(
                    table_hbm.at[idx_v.at[j]], rows[b], gsems[b]).start()
            for b in range(NB):
                j = g * NB + b
                pltpu.make_async_copy(
                    table_hbm.at[idx_v.at[j]], rows[b], gsems[b]).wait()
                pltpu.make_async_copy(
                    rows[b], out_hbm.at[pl.ds(base + j * CHUNK, CHUNK)],
                    ssems[b]).start()
            for b in range(NB):
                j = g * NB + b
                pltpu.make_async_copy(
                    rows[b], out_hbm.at[pl.ds(base + j * CHUNK, CHUNK)],
                    ssems[b]).wait()
            return carry

        lax.fori_loop(0, n_groups, group, 0)

    return k


def kernel(indices, weight):
    batch, fields = indices.shape
    n_rows = batch * fields
    dim = weight.shape[1]
    assert n_rows % (NW * CHUNK * NB) == 0
    n_chunks_w = n_rows // (NW * CHUNK)
    idx3 = indices.astype(jnp.int32).reshape(NW, n_chunks_w, CHUNK)
    out = _make_kernel(n_rows, dim, n_chunks_w)(idx3, weight)
    return out.reshape(batch, fields, dim)
